# baseline (speedup 1.0000x reference)
import functools

import jax
import jax.numpy as jnp
import numpy as np
from jax import lax
from jax.experimental import pallas as pl
from jax.experimental.pallas import tpu as pltpu

B, N, K, H, IN, L, OUT = 4, 512, 30, 128, 64, 3, 400
NPOS, NRBF = 16, 16
CH = 128
NCH = N // CH
EC = CH * K
E = N * K


def _ln(x):
    m = jnp.mean(x, -1, keepdims=True)
    v = jnp.mean((x - m) ** 2, -1, keepdims=True)
    return (x - m) / jnp.sqrt(v + 1e-5)


def _relu(x):
    return jnp.maximum(x, 0.0)


def _mmf(a, b):
    return lax.dot_general(a, b, (((a.ndim - 1,), (0,)), ((), ())),
                           preferred_element_type=jnp.float32)


def _bf(x):
    return x.astype(jnp.bfloat16)


def _embed_body(vfeat_ref, vemb_ref, efeat_ref, enb_ref,
                wn_ref, bn_ref, wv_ref, bv_ref,
                wef_ref, bef_ref, we_ref, be_ref,
                hv_ref, he_ref):
    vf = _ln(_mmf(_bf(vfeat_ref[0]), wn_ref[:]) + bn_ref[:])
    hv_ref[0] = (_mmf(_bf(vf), wv_ref[0:H])
                 + _mmf(_bf(vemb_ref[0]), wv_ref[H:H + IN]) + bv_ref[:])
    ef = _ln(_mmf(_bf(efeat_ref[0]), wef_ref[:]) + bef_ref[:])
    he_ref[0] = (_mmf(_bf(ef), we_ref[0:H])
                 + _mmf(_bf(enb_ref[0]), we_ref[H:H + IN]) + be_ref[:])


def _layer_body(hv_ref, he_ref, eidx_ref, idx0_ref,
                w1e_ref, b1e_ref, w2e_ref, b2e_ref, w3e_ref, b3e_ref,
                f1e_ref, f1be_ref, f2e_ref, f2be_ref,
                w1n_ref, b1n_ref, w2n_ref, b2n_ref, w3n_ref, b3n_ref,
                heo_ref, dh_ref):
    c = pl.program_id(1)
    hv = _bf(hv_ref[0])
    he = he_ref[0]
    idx = eidx_ref[0]
    i0 = idx0_ref[0]
    oh = _bf(idx == lax.broadcasted_iota(jnp.int32, (EC, N), 1))
    hj = _bf(_mmf(oh, hv))
    oh0 = _bf(i0 == lax.broadcasted_iota(jnp.int32, (CH, N), 1))
    hin = _bf(_mmf(oh0, hv))
    rep = _bf(lax.broadcasted_iota(jnp.int32, (EC, CH), 1)
              == lax.broadcasted_iota(jnp.int32, (EC, CH), 0) // K)
    hi = _bf(_mmf(rep, hin))
    w1 = w1e_ref[:]
    m = _relu(_mmf(hi, w1[0:H]) + _mmf(hj, w1[H:2 * H])
              + _mmf(_bf(he), w1[2 * H:3 * H]) + b1e_ref[:])
    m = _relu(_mmf(_bf(m), w2e_ref[:]) + b2e_ref[:])
    m = _mmf(_bf(m), w3e_ref[:]) + b3e_ref[:]
    he = _ln(he + m)
    ff = _relu(_mmf(_bf(he), f1e_ref[:]) + f1be_ref[:])
    ff = _mmf(_bf(ff), f2e_ref[:]) + f2be_ref[:]
    he = _ln(he + ff)
    heo_ref[0] = he
    hvc = _bf(hv_ref[0, pl.ds(c * CH, CH), :])
    hself = _bf(_mmf(rep, hvc))
    w1n = w1n_ref[:]
    m2 = _relu(_mmf(hself, w1n[0:H]) + _mmf(hj, w1n[H:2 * H])
               + _mmf(_bf(he), w1n[2 * H:3 * H]) + b1n_ref[:])
    m2 = _relu(_mmf(_bf(m2), w2n_ref[:]) + b2n_ref[:])
    m2 = _mmf(_bf(m2), w3n_ref[:]) + b3n_ref[:]
    dh_ref[0] = lax.dot_general(
        rep, _bf(m2), (((0,), (0,)), ((), ())),
        preferred_element_type=jnp.float32) * (1.0 / 30.0)


def _hvupd_body(hv_ref, dh_ref, f1_ref, f1b_ref, f2_ref, f2b_ref, out_ref):
    hv = _ln(hv_ref[0] + dh_ref[0])
    ff = _relu(_mmf(_bf(hv), f1_ref[:]) + f1b_ref[:])
    ff = _mmf(_bf(ff), f2_ref[:]) + f2b_ref[:]
    out_ref[0] = _ln(hv + ff)


def _outmerge_body(he_ref, rev_ref, ex_ref, w_ref, b_ref, wp_ref, bp_ref,
                   out_ref):
    a = _mmf(_bf(he_ref[0]), w_ref[:]) + b_ref[:]
    bt = _mmf(_bf(rev_ref[0]), wp_ref[:]) + bp_ref[:]
    ex = ex_ref[0]
    out_ref[0] = a + 0.5 * ex * (bt - a)


def _edge_spec(d, dtype=None):
    return pl.BlockSpec((1, EC, d), lambda b, c: (b, c, 0))


def _full_spec(shape):
    nd = len(shape)
    return pl.BlockSpec(shape, lambda b, c, _nd=nd: (0,) * _nd)


def kernel(V_embed, E_embed, X, x_mask, chain_idx, params):
    p = params
    f32 = jnp.float32

    X_ca = X[:, :, 1, :]
    m2 = x_mask[:, :, None] * x_mask[:, None, :]
    dX = X_ca[:, None, :, :] - X_ca[:, :, None, :]
    D = jnp.sqrt(jnp.sum(dX ** 2, -1) + 1e-6)
    D_max = jnp.max(D, axis=-1, keepdims=True)
    D_adj = D + (1.0 - m2) * D_max
    negD, E_idx = jax.lax.top_k(-D_adj, K)
    D_nb = -negD
    mu = jnp.linspace(0.0, 20.0, NRBF)
    sigma = 20.0 / NRBF
    RBF = jnp.exp(-(((D_nb[..., None] - mu) / sigma) ** 2))
    Xb = X[:, :, :3, :].reshape(B, 3 * N, 3)
    dXb = Xb[:, 1:] - Xb[:, :-1]
    U = dXb / (jnp.linalg.norm(dXb, axis=-1, keepdims=True) + 1e-7)
    u2, u1, u0 = U[:, :-2], U[:, 1:-1], U[:, 2:]
    n2 = jnp.cross(u2, u1)
    n2 = n2 / (jnp.linalg.norm(n2, axis=-1, keepdims=True) + 1e-7)
    n1 = jnp.cross(u1, u0)
    n1 = n1 / (jnp.linalg.norm(n1, axis=-1, keepdims=True) + 1e-7)
    cosD = jnp.clip(jnp.sum(n2 * n1, -1), -1.0 + 1e-7, 1.0 - 1e-7)
    ang = jnp.sign(jnp.sum(u2 * n1, -1)) * jnp.arccos(cosD)
    ang = jnp.pad(ang, ((0, 0), (1, 2))).reshape(B, N, 3)
    Vfeat = jnp.concatenate([jnp.cos(ang), jnp.sin(ang)], -1)
    ii = jnp.arange(N)
    chain_j = jax.vmap(lambda n, i: n[i])(chain_idx, E_idx)
    same = (chain_j == chain_idx[:, :, None]).astype(f32)
    off = (E_idx - ii[None, :, None]).astype(f32) * same
    freq = jnp.exp(jnp.arange(0, NPOS, 2).astype(f32) * (-np.log(10000.0) / NPOS))
    a = off[..., None] * freq
    Epos = jnp.concatenate([jnp.cos(a), jnp.sin(a)], -1)
    Efeat_raw = jnp.concatenate([Epos, RBF], -1).reshape(B, E, NPOS + NRBF)

    E_nb = jnp.take_along_axis(E_embed, E_idx[..., None], axis=2).reshape(B, E, IN)

    E_nbnb = jax.vmap(lambda e: e[e])(E_idx)
    match = E_nbnb == jnp.arange(N)[None, :, None, None]
    rev_exists = jnp.any(match, -1)
    rev_k = jnp.argmax(match, -1).astype(jnp.int32)

    eidx_flat = E_idx.reshape(B, E, 1).astype(jnp.int32)
    idx0 = E_idx[:, :, 0:1].astype(jnp.int32)
    Vfeat_pad = jnp.pad(Vfeat, ((0, 0), (0, 0), (0, 2)))
    wn_pad = jnp.pad(p["feat_Wn_w"], ((0, 2), (0, 0)))

    def b2(x):
        return x.reshape(1, -1)

    grid = (B, NCH)
    hv, he = pl.pallas_call(
        _embed_body,
        grid=grid,
        in_specs=[
            pl.BlockSpec((1, CH, 8), lambda b, c: (b, c, 0)),
            pl.BlockSpec((1, CH, IN), lambda b, c: (b, c, 0)),
            _edge_spec(NPOS + NRBF),
            _edge_spec(IN),
            _full_spec((8, H)), _full_spec((1, H)),
            _full_spec((H + IN, H)), _full_spec((1, H)),
            _full_spec((NPOS + NRBF, H)), _full_spec((1, H)),
            _full_spec((H + IN, H)), _full_spec((1, H)),
        ],
        out_specs=[
            pl.BlockSpec((1, CH, H), lambda b, c: (b, c, 0)),
            _edge_spec(H),
        ],
        out_shape=[
            jax.ShapeDtypeStruct((B, N, H), f32),
            jax.ShapeDtypeStruct((B, E, H), f32),
        ],
    )(Vfeat_pad, V_embed, Efeat_raw, E_nb,
      _bf(wn_pad), b2(p["feat_Wn_b"]), _bf(p["Wv_w"]), b2(p["Wv_b"]),
      _bf(p["feat_We_w"]), b2(p["feat_We_b"]), _bf(p["We_w"]), b2(p["We_b"]))

    layer_call = pl.pallas_call(
        _layer_body,
        grid=grid,
        in_specs=[
            pl.BlockSpec((1, N, H), lambda b, c: (b, 0, 0)),
            _edge_spec(H),
            pl.BlockSpec((1, EC, 1), lambda b, c: (b, c, 0)),
            pl.BlockSpec((1, CH, 1), lambda b, c: (b, c, 0)),
            _full_spec((3 * H, H)), _full_spec((1, H)),
            _full_spec((H, H)), _full_spec((1, H)),
            _full_spec((H, H)), _full_spec((1, H)),
            _full_spec((H, 4 * H)), _full_spec((1, 4 * H)),
            _full_spec((4 * H, H)), _full_spec((1, H)),
            _full_spec((3 * H, H)), _full_spec((1, H)),
            _full_spec((H, H)), _full_spec((1, H)),
            _full_spec((H, H)), _full_spec((1, H)),
        ],
        out_specs=[
            _edge_spec(H),
            pl.BlockSpec((1, CH, H), lambda b, c: (b, c, 0)),
        ],
        out_shape=[
            jax.ShapeDtypeStruct((B, E, H), f32),
            jax.ShapeDtypeStruct((B, N, H), f32),
        ],
    )

    hvupd_call = pl.pallas_call(
        _hvupd_body,
        grid=(B,),
        in_specs=[
            pl.BlockSpec((1, N, H), lambda b: (b, 0, 0)),
            pl.BlockSpec((1, N, H), lambda b: (b, 0, 0)),
            pl.BlockSpec((H, 4 * H), lambda b: (0, 0)),
            pl.BlockSpec((1, 4 * H), lambda b: (0, 0)),
            pl.BlockSpec((4 * H, H), lambda b: (0, 0)),
            pl.BlockSpec((1, H), lambda b: (0, 0)),
        ],
        out_specs=pl.BlockSpec((1, N, H), lambda b: (b, 0, 0)),
        out_shape=jax.ShapeDtypeStruct((B, N, H), f32),
    )

    for l in range(L):
        he, dh = layer_call(
            hv, he, eidx_flat, idx0,
            _bf(p["eW1"][l]), b2(p["eB1"][l]), _bf(p["eW2"][l]), b2(p["eB2"][l]),
            _bf(p["eW3"][l]), b2(p["eB3"][l]),
            _bf(p["eF1w"][l]), b2(p["eF1b"][l]), _bf(p["eF2w"][l]), b2(p["eF2b"][l]),
            _bf(p["nW1"][l]), b2(p["nB1"][l]), _bf(p["nW2"][l]), b2(p["nB2"][l]),
            _bf(p["nW3"][l]), b2(p["nB3"][l]))
        hv = hvupd_call(hv, dh, _bf(p["nF1w"][l]), b2(p["nF1b"][l]),
                        _bf(p["nF2w"][l]), b2(p["nF2b"][l]))

    perm = np.arange(OUT).reshape(20, 20).T.reshape(-1)
    flat_j = (E_idx * K + rev_k).reshape(B, E)
    rev_he = jax.vmap(lambda t, i: t[i])(he, flat_j)
    ex = rev_exists.astype(f32).reshape(B, E, 1)

    out = pl.pallas_call(
        _outmerge_body,
        grid=grid,
        in_specs=[
            _edge_spec(H),
            _edge_spec(H),
            _edge_spec(1),
            _full_spec((H, OUT)),
            _full_spec((1, OUT)),
            _full_spec((H, OUT)),
            _full_spec((1, OUT)),
        ],
        out_specs=_edge_spec(OUT),
        out_shape=jax.ShapeDtypeStruct((B, E, OUT), f32),
    )(he, rev_he, ex, _bf(p["Wout_w"]), b2(p["Wout_b"]),
      _bf(p["Wout_w"][:, perm]), b2(p["Wout_b"][perm]))

    return out.reshape(B, N, K, OUT), E_idx


# speedup vs baseline: 1.9055x; 1.9055x over previous
import functools

import jax
import jax.numpy as jnp
import numpy as np
from jax import lax
from jax.experimental import pallas as pl
from jax.experimental.pallas import tpu as pltpu

B, N, K, H, IN, L, OUT = 4, 512, 30, 128, 64, 3, 400
NPOS, NRBF = 16, 16
CH = 128
NCH = N // CH
EC = CH * K
E = N * K


def _ln(x):
    m = jnp.mean(x, -1, keepdims=True)
    v = jnp.mean((x - m) ** 2, -1, keepdims=True)
    return (x - m) / jnp.sqrt(v + 1e-5)


def _relu(x):
    return jnp.maximum(x, 0.0)


def _mmf(a, b):
    return lax.dot_general(a, b, (((a.ndim - 1,), (0,)), ((), ())),
                           preferred_element_type=jnp.float32)


def _bf(x):
    return x.astype(jnp.bfloat16)


def _embed_body(vfeat_ref, vemb_ref, dnb_ref, eidx_ref, chain_ref, enb_ref,
                wn_ref, bn_ref, wv_ref, bv_ref,
                wef_ref, bef_ref, we_ref, be_ref,
                hv_ref, he_ref):
    f32 = jnp.float32
    c = pl.program_id(1)
    vf = _ln(_mmf(_bf(vfeat_ref[0]), wn_ref[:]) + bn_ref[:])
    hv_ref[0] = (_mmf(_bf(vf), wv_ref[0:H])
                 + _mmf(_bf(vemb_ref[0]), wv_ref[H:H + IN]) + bv_ref[:])
    idx = eidx_ref[0]
    oh = _bf(idx == lax.broadcasted_iota(jnp.int32, (EC, N), 1))
    rep = _bf(lax.broadcasted_iota(jnp.int32, (EC, CH), 1)
              == lax.broadcasted_iota(jnp.int32, (EC, CH), 0) // K)
    chainf = chain_ref[0].astype(f32)
    cj = _mmf(oh, _bf(chainf))
    ci = _mmf(rep, _bf(chain_ref[0, pl.ds(c * CH, CH), :].astype(f32)))
    same = (cj == ci).astype(f32)
    i_self = (lax.broadcasted_iota(jnp.int32, (EC, 1), 0) // K) + c * CH
    off = (idx - i_self).astype(f32) * same
    freq = jnp.exp(lax.broadcasted_iota(jnp.int32, (1, NPOS // 2), 1).astype(f32)
                   * (2.0 * (-np.log(10000.0) / NPOS)))
    av = off * freq
    mu = (lax.broadcasted_iota(jnp.int32, (1, NRBF), 1).astype(f32)
          * (20.0 / (NRBF - 1)))
    rbf = jnp.exp(-(((dnb_ref[0] - mu) / (20.0 / NRBF)) ** 2))
    efeat = jnp.concatenate([jnp.cos(av), jnp.sin(av), rbf], -1)
    ef = _ln(_mmf(_bf(efeat), wef_ref[:]) + bef_ref[:])
    he_ref[0] = (_mmf(_bf(ef), we_ref[0:H])
                 + _mmf(_bf(enb_ref[0]), we_ref[H:H + IN]) + be_ref[:])


def _rtab_body(eidx_ref, r_ref):
    idx = eidx_ref[0]
    oh = _bf(idx == lax.broadcasted_iota(jnp.int32, (EC, N), 1))
    i_local = lax.broadcasted_iota(jnp.int32, (EC, CH), 0)
    repk = _bf(jnp.where(
        lax.broadcasted_iota(jnp.int32, (EC, CH), 1) == i_local // K,
        (i_local % K + 1).astype(jnp.float32), 0.0))
    r_ref[0] = lax.dot_general(repk, oh, (((0,), (0,)), ((), ())),
                               preferred_element_type=jnp.float32)


def _revk_body(eidx_ref, r_ref, fj_ref, ex_ref):
    idx = eidx_ref[0]
    oh = _bf(idx == lax.broadcasted_iota(jnp.int32, (EC, N), 1))
    t = _mmf(oh, _bf(r_ref[0]))
    rep = (lax.broadcasted_iota(jnp.int32, (EC, CH), 1)
           == lax.broadcasted_iota(jnp.int32, (EC, CH), 0) // K
           ).astype(jnp.float32)
    rv = jnp.sum(t * rep, axis=1, keepdims=True)
    ex_ref[0] = (rv > 0.0).astype(jnp.float32)
    rk = jnp.maximum(rv - 1.0, 0.0).astype(jnp.int32)
    fj_ref[0] = idx * K + rk


def _layer_body(hv_ref, he_ref, eidx_ref, idx0_ref,
                w1e_ref, b1e_ref, w2e_ref, b2e_ref, w3e_ref, b3e_ref,
                f1e_ref, f1be_ref, f2e_ref, f2be_ref,
                w1n_ref, b1n_ref, w2n_ref, b2n_ref, w3n_ref, b3n_ref,
                heo_ref, dh_ref):
    c = pl.program_id(1)
    hv = _bf(hv_ref[0])
    he = he_ref[0]
    idx = eidx_ref[0]
    i0 = idx0_ref[0]
    oh = _bf(idx == lax.broadcasted_iota(jnp.int32, (EC, N), 1))
    hj = _bf(_mmf(oh, hv))
    oh0 = _bf(i0 == lax.broadcasted_iota(jnp.int32, (CH, N), 1))
    hin = _bf(_mmf(oh0, hv))
    rep = _bf(lax.broadcasted_iota(jnp.int32, (EC, CH), 1)
              == lax.broadcasted_iota(jnp.int32, (EC, CH), 0) // K)
    hi = _bf(_mmf(rep, hin))
    w1 = w1e_ref[:]
    m = _relu(_mmf(hi, w1[0:H]) + _mmf(hj, w1[H:2 * H])
              + _mmf(_bf(he), w1[2 * H:3 * H]) + b1e_ref[:])
    m = _relu(_mmf(_bf(m), w2e_ref[:]) + b2e_ref[:])
    m = _mmf(_bf(m), w3e_ref[:]) + b3e_ref[:]
    he = _ln(he + m)
    ff = _relu(_mmf(_bf(he), f1e_ref[:]) + f1be_ref[:])
    ff = _mmf(_bf(ff), f2e_ref[:]) + f2be_ref[:]
    he = _ln(he + ff)
    heo_ref[0] = he
    hvc = _bf(hv_ref[0, pl.ds(c * CH, CH), :])
    hself = _bf(_mmf(rep, hvc))
    w1n = w1n_ref[:]
    m2 = _relu(_mmf(hself, w1n[0:H]) + _mmf(hj, w1n[H:2 * H])
               + _mmf(_bf(he), w1n[2 * H:3 * H]) + b1n_ref[:])
    m2 = _relu(_mmf(_bf(m2), w2n_ref[:]) + b2n_ref[:])
    m2 = _mmf(_bf(m2), w3n_ref[:]) + b3n_ref[:]
    dh_ref[0] = lax.dot_general(
        rep, _bf(m2), (((0,), (0,)), ((), ())),
        preferred_element_type=jnp.float32) * (1.0 / 30.0)


def _hvupd_body(hv_ref, dh_ref, f1_ref, f1b_ref, f2_ref, f2b_ref, out_ref):
    hv = _ln(hv_ref[0] + dh_ref[0])
    ff = _relu(_mmf(_bf(hv), f1_ref[:]) + f1b_ref[:])
    ff = _mmf(_bf(ff), f2_ref[:]) + f2b_ref[:]
    out_ref[0] = _ln(hv + ff)


def _outmerge_body(he_ref, rev_ref, ex_ref, w_ref, b_ref, wp_ref, bp_ref,
                   out_ref):
    a = _mmf(_bf(he_ref[0]), w_ref[:]) + b_ref[:]
    bt = _mmf(_bf(rev_ref[0]), wp_ref[:]) + bp_ref[:]
    ex = ex_ref[0]
    out_ref[0] = a + 0.5 * ex * (bt - a)


def _edge_spec(d, dtype=None):
    return pl.BlockSpec((1, EC, d), lambda b, c: (b, c, 0))


def _full_spec(shape):
    nd = len(shape)
    return pl.BlockSpec(shape, lambda b, c, _nd=nd: (0,) * _nd)


def kernel(V_embed, E_embed, X, x_mask, chain_idx, params):
    p = params
    f32 = jnp.float32

    X_ca = X[:, :, 1, :]
    m2 = x_mask[:, :, None] * x_mask[:, None, :]
    dX = X_ca[:, None, :, :] - X_ca[:, :, None, :]
    D = jnp.sqrt(jnp.sum(dX ** 2, -1) + 1e-6)
    D_max = jnp.max(D, axis=-1, keepdims=True)
    D_adj = D + (1.0 - m2) * D_max
    negD, E_idx = jax.lax.top_k(-D_adj, K)
    Xb = X[:, :, :3, :].reshape(B, 3 * N, 3)
    dXb = Xb[:, 1:] - Xb[:, :-1]
    U = dXb / (jnp.linalg.norm(dXb, axis=-1, keepdims=True) + 1e-7)
    u2, u1, u0 = U[:, :-2], U[:, 1:-1], U[:, 2:]
    n2 = jnp.cross(u2, u1)
    n2 = n2 / (jnp.linalg.norm(n2, axis=-1, keepdims=True) + 1e-7)
    n1 = jnp.cross(u1, u0)
    n1 = n1 / (jnp.linalg.norm(n1, axis=-1, keepdims=True) + 1e-7)
    cosD = jnp.clip(jnp.sum(n2 * n1, -1), -1.0 + 1e-7, 1.0 - 1e-7)
    ang = jnp.sign(jnp.sum(u2 * n1, -1)) * jnp.arccos(cosD)
    ang = jnp.pad(ang, ((0, 0), (1, 2))).reshape(B, N, 3)
    Vfeat = jnp.concatenate([jnp.cos(ang), jnp.sin(ang)], -1)

    E_nb = jnp.take_along_axis(E_embed, E_idx[..., None], axis=2).reshape(B, E, IN)

    eidx_flat = E_idx.reshape(B, E, 1).astype(jnp.int32)
    idx0 = E_idx[:, :, 0:1].astype(jnp.int32)
    dnb = (-negD).reshape(B, E, 1)
    chain3 = chain_idx.reshape(B, N, 1).astype(jnp.int32)
    Vfeat_pad = jnp.pad(Vfeat, ((0, 0), (0, 0), (0, 2)))
    wn_pad = jnp.pad(p["feat_Wn_w"], ((0, 2), (0, 0)))

    def b2(x):
        return x.reshape(1, -1)

    grid = (B, NCH)
    hv, he = pl.pallas_call(
        _embed_body,
        grid=grid,
        in_specs=[
            pl.BlockSpec((1, CH, 8), lambda b, c: (b, c, 0)),
            pl.BlockSpec((1, CH, IN), lambda b, c: (b, c, 0)),
            _edge_spec(1),
            _edge_spec(1),
            pl.BlockSpec((1, N, 1), lambda b, c: (b, 0, 0)),
            _edge_spec(IN),
            _full_spec((8, H)), _full_spec((1, H)),
            _full_spec((H + IN, H)), _full_spec((1, H)),
            _full_spec((NPOS + NRBF, H)), _full_spec((1, H)),
            _full_spec((H + IN, H)), _full_spec((1, H)),
        ],
        out_specs=[
            pl.BlockSpec((1, CH, H), lambda b, c: (b, c, 0)),
            _edge_spec(H),
        ],
        out_shape=[
            jax.ShapeDtypeStruct((B, N, H), f32),
            jax.ShapeDtypeStruct((B, E, H), f32),
        ],
    )(Vfeat_pad, V_embed, dnb, eidx_flat, chain3, E_nb,
      _bf(wn_pad), b2(p["feat_Wn_b"]), _bf(p["Wv_w"]), b2(p["Wv_b"]),
      _bf(p["feat_We_w"]), b2(p["feat_We_b"]), _bf(p["We_w"]), b2(p["We_b"]))

    layer_call = pl.pallas_call(
        _layer_body,
        grid=grid,
        in_specs=[
            pl.BlockSpec((1, N, H), lambda b, c: (b, 0, 0)),
            _edge_spec(H),
            pl.BlockSpec((1, EC, 1), lambda b, c: (b, c, 0)),
            pl.BlockSpec((1, CH, 1), lambda b, c: (b, c, 0)),
            _full_spec((3 * H, H)), _full_spec((1, H)),
            _full_spec((H, H)), _full_spec((1, H)),
            _full_spec((H, H)), _full_spec((1, H)),
            _full_spec((H, 4 * H)), _full_spec((1, 4 * H)),
            _full_spec((4 * H, H)), _full_spec((1, H)),
            _full_spec((3 * H, H)), _full_spec((1, H)),
            _full_spec((H, H)), _full_spec((1, H)),
            _full_spec((H, H)), _full_spec((1, H)),
        ],
        out_specs=[
            _edge_spec(H),
            pl.BlockSpec((1, CH, H), lambda b, c: (b, c, 0)),
        ],
        out_shape=[
            jax.ShapeDtypeStruct((B, E, H), f32),
            jax.ShapeDtypeStruct((B, N, H), f32),
        ],
    )

    hvupd_call = pl.pallas_call(
        _hvupd_body,
        grid=(B,),
        in_specs=[
            pl.BlockSpec((1, N, H), lambda b: (b, 0, 0)),
            pl.BlockSpec((1, N, H), lambda b: (b, 0, 0)),
            pl.BlockSpec((H, 4 * H), lambda b: (0, 0)),
            pl.BlockSpec((1, 4 * H), lambda b: (0, 0)),
            pl.BlockSpec((4 * H, H), lambda b: (0, 0)),
            pl.BlockSpec((1, H), lambda b: (0, 0)),
        ],
        out_specs=pl.BlockSpec((1, N, H), lambda b: (b, 0, 0)),
        out_shape=jax.ShapeDtypeStruct((B, N, H), f32),
    )

    for l in range(L):
        he, dh = layer_call(
            hv, he, eidx_flat, idx0,
            _bf(p["eW1"][l]), b2(p["eB1"][l]), _bf(p["eW2"][l]), b2(p["eB2"][l]),
            _bf(p["eW3"][l]), b2(p["eB3"][l]),
            _bf(p["eF1w"][l]), b2(p["eF1b"][l]), _bf(p["eF2w"][l]), b2(p["eF2b"][l]),
            _bf(p["nW1"][l]), b2(p["nB1"][l]), _bf(p["nW2"][l]), b2(p["nB2"][l]),
            _bf(p["nW3"][l]), b2(p["nB3"][l]))
        hv = hvupd_call(hv, dh, _bf(p["nF1w"][l]), b2(p["nF1b"][l]),
                        _bf(p["nF2w"][l]), b2(p["nF2b"][l]))

    R = pl.pallas_call(
        _rtab_body,
        grid=grid,
        in_specs=[_edge_spec(1)],
        out_specs=pl.BlockSpec((1, CH, N), lambda b, c: (b, c, 0)),
        out_shape=jax.ShapeDtypeStruct((B, N, N), f32),
    )(eidx_flat)
    flat_j3, ex = pl.pallas_call(
        _revk_body,
        grid=grid,
        in_specs=[
            _edge_spec(1),
            pl.BlockSpec((1, N, CH), lambda b, c: (b, 0, c)),
        ],
        out_specs=[_edge_spec(1), _edge_spec(1)],
        out_shape=[
            jax.ShapeDtypeStruct((B, E, 1), jnp.int32),
            jax.ShapeDtypeStruct((B, E, 1), f32),
        ],
    )(eidx_flat, R)

    perm = np.arange(OUT).reshape(20, 20).T.reshape(-1)
    flat_j = flat_j3.reshape(B, E)
    rev_he = jax.vmap(lambda t, i: t[i])(he, flat_j)

    out = pl.pallas_call(
        _outmerge_body,
        grid=grid,
        in_specs=[
            _edge_spec(H),
            _edge_spec(H),
            _edge_spec(1),
            _full_spec((H, OUT)),
            _full_spec((1, OUT)),
            _full_spec((H, OUT)),
            _full_spec((1, OUT)),
        ],
        out_specs=_edge_spec(OUT),
        out_shape=jax.ShapeDtypeStruct((B, E, OUT), f32),
    )(he, rev_he, ex, _bf(p["Wout_w"]), b2(p["Wout_b"]),
      _bf(p["Wout_w"][:, perm]), b2(p["Wout_b"][perm]))

    return out.reshape(B, N, K, OUT), E_idx
